# baseline (speedup 1.0000x reference)
import functools

import jax
import jax.numpy as jnp
from jax import lax
from jax.experimental import pallas as pl
from jax.experimental.pallas import tpu as pltpu
from jax.experimental.pallas import tpu_sc as plsc

N = 50000
E = 400000
D = 128
NPAD = 50176
ROW_BLK = 512
GRID = NPAD // ROW_BLK

NC, NS, L = 2, 16, 16
CB = D // L
RT = NPAD // NS
GB = 256
K = 5
HK = 4
EPT = 25600
NB = EPT // GB
EP = EPT * NS


def _sc_agg_body(srcs, dsts, ones_hbm, zeros_hbm, hv_hbm,
                 agg0_out, agg1_out, deg0_out, deg1_out,
                 accum, src_v, dst_v, idxb, bufs, ones_v,
                 gsem, ssem):
    core = lax.axis_index("c")
    sid = lax.axis_index("s")
    w = core * NS + sid
    r0 = sid * RT

    pltpu.sync_copy(srcs.at[w], src_v)
    pltpu.sync_copy(dsts.at[w], dst_v)
    pltpu.sync_copy(ones_hbm, ones_v)

    for c in range(CB + 1):
        pltpu.sync_copy(zeros_hbm, accum.at[pl.ds(r0, RT)])
        plsc.subcore_barrier()

        if c < CB:
            def gather(i, s):
                for v in range(GB // L):
                    idxb[s, pl.ds(v * L, L)] = (
                        src_v[pl.ds(i * GB + v * L, L)] + c)
                pltpu.async_copy(hv_hbm.at[idxb.at[s]],
                                 bufs.at[s], gsem.at[s])

            def dwait(sem, s):
                pltpu.make_async_copy(hv_hbm.at[pl.ds(0, GB)], bufs.at[s],
                                      sem.at[s]).wait()

            for s in range(HK):
                gather(s, s)

            def batch_body(j, carry):
                base = j * K
                for s in range(K):
                    i = base + s
                    sf = (s + HK) % K
                    dwait(gsem, s)
                    pltpu.sync_copy(bufs.at[s], accum.at[dst_v.at[i]],
                                    add=True)
                    jf = i + HK

                    @pl.when(jf < NB)
                    def _():
                        gather(jf, sf)
                return carry

            lax.fori_loop(0, NB // K, batch_body, 0)
        else:
            def deg_body(j, carry):
                base = j * K
                for s in range(K):
                    pltpu.sync_copy(ones_v, accum.at[dst_v.at[base + s]],
                                    add=True)
                return carry

            lax.fori_loop(0, NB // K, deg_body, 0)

        plsc.subcore_barrier()
        if c < CB:
            @pl.when(core == 0)
            def _():
                pltpu.sync_copy(accum.at[pl.ds(r0, RT)],
                                agg0_out.at[pl.ds(r0, RT), pl.ds(c * L, L)])

            @pl.when(core == 1)
            def _():
                pltpu.sync_copy(accum.at[pl.ds(r0, RT)],
                                agg1_out.at[pl.ds(r0, RT), pl.ds(c * L, L)])
        else:
            @pl.when(core == 0)
            def _():
                pltpu.sync_copy(accum.at[pl.ds(r0, RT)],
                                deg0_out.at[pl.ds(r0, RT)])

            @pl.when(core == 1)
            def _():
                pltpu.sync_copy(accum.at[pl.ds(r0, RT)],
                                deg1_out.at[pl.ds(r0, RT)])


@jax.jit
def _sc_agg(srcs, dsts, ones, zeros, hv):
    mesh = plsc.VectorSubcoreMesh(core_axis_name="c", subcore_axis_name="s")
    f = pl.kernel(
        _sc_agg_body,
        out_type=(
            jax.ShapeDtypeStruct((NPAD, D), jnp.float32),
            jax.ShapeDtypeStruct((NPAD, D), jnp.float32),
            jax.ShapeDtypeStruct((NPAD, L), jnp.float32),
            jax.ShapeDtypeStruct((NPAD, L), jnp.float32),
        ),
        mesh=mesh,
        compiler_params=pltpu.CompilerParams(use_tc_tiling_on_sc=False),
        scratch_types=[
            pltpu.VMEM_SHARED((NPAD, L), jnp.float32),
            pltpu.VMEM((EPT,), jnp.int32),
            pltpu.VMEM((NB, GB), jnp.int32),
            pltpu.VMEM((K, GB), jnp.int32),
            pltpu.VMEM((K, GB, L), jnp.float32),
            pltpu.VMEM((GB, L), jnp.float32),
            pltpu.SemaphoreType.DMA((K,)),
            pltpu.SemaphoreType.DMA((K,)),
        ],
    )
    return f(srcs, dsts, ones, zeros, hv)


def _self_tc_body(h_ref, ws0_ref, ws1_ref, b0_ref, b1_ref, out_ref):
    wsum = ws0_ref[...] + ws1_ref[...]
    bsum = b0_ref[...] + b1_ref[...]
    out_ref[...] = jnp.dot(h_ref[...], wsum,
                           preferred_element_type=jnp.float32) + bsum


@jax.jit
def _self_tc(h, Ws0, Ws1, b0, b1):
    row_spec = pl.BlockSpec((ROW_BLK, D), lambda i: (i, 0))
    w_spec = pl.BlockSpec((D, D), lambda i: (0, 0))
    b_spec = pl.BlockSpec((1, D), lambda i: (0, 0))
    return pl.pallas_call(
        _self_tc_body,
        grid=(GRID,),
        in_specs=[row_spec, w_spec, w_spec, b_spec, b_spec],
        out_specs=row_spec,
        out_shape=jax.ShapeDtypeStruct((N, D), jnp.float32),
    )(h, Ws0, Ws1, b0.reshape(1, D), b1.reshape(1, D))


def _sage_tc_body(self_ref, agg0_ref, agg1_ref, deg0_ref, deg1_ref,
                  wn0_ref, wn1_ref, out_ref):
    rd0 = 1.0 / jnp.clip(deg0_ref[...][:, 0:1], 1.0, None)
    rd1 = 1.0 / jnp.clip(deg1_ref[...][:, 0:1], 1.0, None)
    mean0 = agg0_ref[...] * rd0
    mean1 = agg1_ref[...] * rd1
    acc = self_ref[...]
    acc += jnp.dot(mean0, wn0_ref[...], preferred_element_type=jnp.float32)
    acc += jnp.dot(mean1, wn1_ref[...], preferred_element_type=jnp.float32)
    out_ref[...] = acc


@jax.jit
def _sage_tc(selfp, agg0, agg1, deg0, deg1, Wn0, Wn1):
    row_spec = pl.BlockSpec((ROW_BLK, D), lambda i: (i, 0))
    deg_spec = pl.BlockSpec((ROW_BLK, 16), lambda i: (i, 0))
    w_spec = pl.BlockSpec((D, D), lambda i: (0, 0))
    return pl.pallas_call(
        _sage_tc_body,
        grid=(GRID,),
        in_specs=[row_spec, row_spec, row_spec, deg_spec, deg_spec,
                  w_spec, w_spec],
        out_specs=row_spec,
        out_shape=jax.ShapeDtypeStruct((N, D), jnp.float32),
    )(selfp, agg0, agg1, deg0, deg1, Wn0, Wn1)


def _prep_edges(edge_index):
    src = edge_index[0].astype(jnp.int32)
    dst = edge_index[1].astype(jnp.int32)
    npad = EP - E
    ar = jnp.arange(npad, dtype=jnp.int32)
    pad_src = ar & 32767
    pad_dst = N + (ar & 127)
    src_p = (jnp.concatenate([src, pad_src]) * CB).reshape(NS, EPT)
    dst_p = jnp.concatenate([dst, pad_dst]).reshape(NS, NB, GB)
    return src_p, dst_p


def kernel(h, edge_index_r0, edge_index_r1, W_self_r0, W_neigh_r0, b_r0,
           W_self_r1, W_neigh_r1, b_r1):
    s0, d0 = _prep_edges(edge_index_r0)
    s1, d1 = _prep_edges(edge_index_r1)
    srcs = jnp.concatenate([s0, s1], axis=0)
    dsts = jnp.concatenate([d0, d1], axis=0)
    ones = jnp.ones((GB, L), jnp.float32)
    zeros = jnp.zeros((RT, L), jnp.float32)

    hv = h.reshape(N * CB, L)
    selfp = _self_tc(h, W_self_r0, W_self_r1, b_r0, b_r1)
    agg0, agg1, deg0, deg1 = _sc_agg(srcs, dsts, ones, zeros, hv)

    return _sage_tc(selfp, agg0, agg1, deg0, deg1, W_neigh_r0, W_neigh_r1)


# speedup vs baseline: 1.0515x; 1.0515x over previous
import functools

import jax
import jax.numpy as jnp
from jax import lax
from jax.experimental import pallas as pl
from jax.experimental.pallas import tpu as pltpu
from jax.experimental.pallas import tpu_sc as plsc

N = 50000
E = 400000
D = 128
NPAD = 50176
ROW_BLK = 1024
GRID = NPAD // ROW_BLK

NC, NS, L = 2, 16, 16
CB = D // L
RT = NPAD // NS
GB = 256
K = 5
HK = 4
EPT = 25600
NB = EPT // GB
EP = EPT * NS


def _sc_agg_body(srcs, dsts, ones_hbm, zeros_hbm, hv_hbm,
                 agg0_out, agg1_out, deg0_out, deg1_out,
                 accum, src_v, dst_v, idxb, bufs, ones_v,
                 gsem, ssem):
    core = lax.axis_index("c")
    sid = lax.axis_index("s")
    w = core * NS + sid
    r0 = sid * RT

    pltpu.sync_copy(srcs.at[w], src_v)
    pltpu.sync_copy(dsts.at[w], dst_v)
    pltpu.sync_copy(ones_hbm, ones_v)

    for c in range(CB + 1):
        pltpu.sync_copy(zeros_hbm, accum.at[pl.ds(r0, RT)])
        plsc.subcore_barrier()

        if c < CB:
            def gather(i, s):
                for v in range(GB // L):
                    idxb[s, pl.ds(v * L, L)] = (
                        src_v[pl.ds(i * GB + v * L, L)] + c)
                pltpu.async_copy(hv_hbm.at[idxb.at[s]],
                                 bufs.at[s], gsem.at[s])

            def dwait(sem, s):
                pltpu.make_async_copy(hv_hbm.at[pl.ds(0, GB)], bufs.at[s],
                                      sem.at[s]).wait()

            for s in range(HK):
                gather(s, s)

            def batch_body(j, carry):
                base = j * K
                for s in range(K):
                    i = base + s
                    sf = (s + HK) % K
                    dwait(gsem, s)
                    pltpu.sync_copy(bufs.at[s], accum.at[dst_v.at[i]],
                                    add=True)
                    jf = i + HK

                    @pl.when(jf < NB)
                    def _():
                        gather(jf, sf)
                return carry

            lax.fori_loop(0, NB // K, batch_body, 0)
        else:
            def deg_body(j, carry):
                base = j * K
                for s in range(K):
                    pltpu.sync_copy(ones_v, accum.at[dst_v.at[base + s]],
                                    add=True)
                return carry

            lax.fori_loop(0, NB // K, deg_body, 0)

        plsc.subcore_barrier()
        if c < CB:
            @pl.when(core == 0)
            def _():
                pltpu.sync_copy(accum.at[pl.ds(r0, RT)],
                                agg0_out.at[pl.ds(r0, RT), pl.ds(c * L, L)])

            @pl.when(core == 1)
            def _():
                pltpu.sync_copy(accum.at[pl.ds(r0, RT)],
                                agg1_out.at[pl.ds(r0, RT), pl.ds(c * L, L)])
        else:
            @pl.when(core == 0)
            def _():
                pltpu.sync_copy(accum.at[pl.ds(r0, RT)],
                                deg0_out.at[pl.ds(r0, RT)])

            @pl.when(core == 1)
            def _():
                pltpu.sync_copy(accum.at[pl.ds(r0, RT)],
                                deg1_out.at[pl.ds(r0, RT)])


@jax.jit
def _sc_agg(srcs, dsts, ones, zeros, hv):
    mesh = plsc.VectorSubcoreMesh(core_axis_name="c", subcore_axis_name="s")
    f = pl.kernel(
        _sc_agg_body,
        out_type=(
            jax.ShapeDtypeStruct((NPAD, D), jnp.float32),
            jax.ShapeDtypeStruct((NPAD, D), jnp.float32),
            jax.ShapeDtypeStruct((NPAD, L), jnp.float32),
            jax.ShapeDtypeStruct((NPAD, L), jnp.float32),
        ),
        mesh=mesh,
        compiler_params=pltpu.CompilerParams(use_tc_tiling_on_sc=False),
        scratch_types=[
            pltpu.VMEM_SHARED((NPAD, L), jnp.float32),
            pltpu.VMEM((EPT,), jnp.int32),
            pltpu.VMEM((NB, GB), jnp.int32),
            pltpu.VMEM((K, GB), jnp.int32),
            pltpu.VMEM((K, GB, L), jnp.float32),
            pltpu.VMEM((GB, L), jnp.float32),
            pltpu.SemaphoreType.DMA((K,)),
            pltpu.SemaphoreType.DMA((K,)),
        ],
    )
    return f(srcs, dsts, ones, zeros, hv)


def _self_tc_body(h_ref, ws0_ref, ws1_ref, b0_ref, b1_ref, out_ref):
    wsum = ws0_ref[...] + ws1_ref[...]
    bsum = b0_ref[...] + b1_ref[...]
    out_ref[...] = jnp.dot(h_ref[...], wsum,
                           preferred_element_type=jnp.float32) + bsum


@jax.jit
def _self_tc(h, Ws0, Ws1, b0, b1):
    row_spec = pl.BlockSpec((ROW_BLK, D), lambda i: (i, 0))
    w_spec = pl.BlockSpec((D, D), lambda i: (0, 0))
    b_spec = pl.BlockSpec((1, D), lambda i: (0, 0))
    return pl.pallas_call(
        _self_tc_body,
        grid=(GRID,),
        in_specs=[row_spec, w_spec, w_spec, b_spec, b_spec],
        out_specs=row_spec,
        out_shape=jax.ShapeDtypeStruct((N, D), jnp.float32),
    )(h, Ws0, Ws1, b0.reshape(1, D), b1.reshape(1, D))


def _sage_tc_body(self_ref, agg0_ref, agg1_ref, deg0_ref, deg1_ref,
                  wn0_ref, wn1_ref, out_ref):
    rd0 = 1.0 / jnp.clip(deg0_ref[...][:, 0:1], 1.0, None)
    rd1 = 1.0 / jnp.clip(deg1_ref[...][:, 0:1], 1.0, None)
    mean0 = agg0_ref[...] * rd0
    mean1 = agg1_ref[...] * rd1
    acc = self_ref[...]
    acc += jnp.dot(mean0, wn0_ref[...], preferred_element_type=jnp.float32)
    acc += jnp.dot(mean1, wn1_ref[...], preferred_element_type=jnp.float32)
    out_ref[...] = acc


@jax.jit
def _sage_tc(selfp, agg0, agg1, deg0, deg1, Wn0, Wn1):
    row_spec = pl.BlockSpec((ROW_BLK, D), lambda i: (i, 0))
    deg_spec = pl.BlockSpec((ROW_BLK, 16), lambda i: (i, 0))
    w_spec = pl.BlockSpec((D, D), lambda i: (0, 0))
    return pl.pallas_call(
        _sage_tc_body,
        grid=(GRID,),
        in_specs=[row_spec, row_spec, row_spec, deg_spec, deg_spec,
                  w_spec, w_spec],
        out_specs=row_spec,
        out_shape=jax.ShapeDtypeStruct((N, D), jnp.float32),
    )(selfp, agg0, agg1, deg0, deg1, Wn0, Wn1)


def _prep_edges(edge_index):
    src = edge_index[0].astype(jnp.int32)
    dst = edge_index[1].astype(jnp.int32)
    npad = EP - E
    ar = jnp.arange(npad, dtype=jnp.int32)
    pad_src = ar & 32767
    pad_dst = N + (ar & 127)
    src_p = (jnp.concatenate([src, pad_src]) * CB).reshape(NS, EPT)
    dst_p = jnp.concatenate([dst, pad_dst]).reshape(NS, NB, GB)
    return src_p, dst_p


def kernel(h, edge_index_r0, edge_index_r1, W_self_r0, W_neigh_r0, b_r0,
           W_self_r1, W_neigh_r1, b_r1):
    s0, d0 = _prep_edges(edge_index_r0)
    s1, d1 = _prep_edges(edge_index_r1)
    srcs = jnp.concatenate([s0, s1], axis=0)
    dsts = jnp.concatenate([d0, d1], axis=0)
    ones = jnp.ones((GB, L), jnp.float32)
    zeros = jnp.zeros((RT, L), jnp.float32)

    hv = h.reshape(N * CB, L)
    selfp = _self_tc(h, W_self_r0, W_self_r1, b_r0, b_r1)
    agg0, agg1, deg0, deg1 = _sc_agg(srcs, dsts, ones, zeros, hv)

    return _sage_tc(selfp, agg0, agg1, deg0, deg1, W_neigh_r0, W_neigh_r1)
